# Initial kernel scaffold; baseline (speedup 1.0000x reference)
import functools

import jax
import jax.numpy as jnp
from jax import lax
from jax.experimental import pallas as pl
from jax.experimental.pallas import tpu as pltpu
from jax.experimental.pallas import tpu_sc as plsc

N_NODES = 10000
N_EDGES = 160000
BATCH = 200
N_PER = 50
D = 256
HALF = 128
MLP_HID = 512
OUT_DIM = 460

NC = 2
NS = 16
E_PAD = 163840
CHUNK = 128
ROWS_PAD = 10240
ROWS_PER_SUB = ROWS_PAD // NS

_sc_mesh = plsc.VectorSubcoreMesh(core_axis_name="c", subcore_axis_name="s")


@functools.partial(
    pl.kernel,
    out_type=jax.ShapeDtypeStruct((NC, ROWS_PAD, 8), jnp.float32),
    mesh=_sc_mesh,
    scratch_types=[
        pltpu.VMEM((CHUNK,), jnp.int32),
        pltpu.VMEM((CHUNK, 8), jnp.float32),
        pltpu.VMEM_SHARED((ROWS_PAD, 8), jnp.float32),
        pltpu.SemaphoreType.DMA,
    ],
)
def _deg_sc(dst_hbm, ones_hbm, zeros8_hbm, out_hbm, dstv, onesv, deg_sh, sem):
    c = lax.axis_index("c")
    s = lax.axis_index("s")
    pltpu.sync_copy(zeros8_hbm, deg_sh.at[pl.ds(s * ROWS_PER_SUB, ROWS_PER_SUB)])
    pltpu.sync_copy(ones_hbm, onesv)
    plsc.subcore_barrier()

    n_chunks = E_PAD // (NC * NS * CHUNK)
    base0 = c * (E_PAD // NC) + s * (n_chunks * CHUNK)

    @pl.loop(0, n_chunks)
    def _(j):
        pltpu.sync_copy(dst_hbm.at[pl.ds(base0 + j * CHUNK, CHUNK)], dstv)
        pltpu.sync_copy(onesv, deg_sh.at[dstv], add=True)

    plsc.subcore_barrier()
    pltpu.sync_copy(
        deg_sh.at[pl.ds(s * ROWS_PER_SUB, ROWS_PER_SUB)],
        out_hbm.at[c].at[pl.ds(s * ROWS_PER_SUB, ROWS_PER_SUB)],
    )


@functools.partial(
    pl.kernel,
    out_type=jax.ShapeDtypeStruct((NC, ROWS_PAD, HALF), jnp.float32),
    mesh=_sc_mesh,
    scratch_types=[
        pltpu.VMEM((CHUNK,), jnp.int32),
        pltpu.VMEM((CHUNK,), jnp.int32),
        pltpu.VMEM((CHUNK, HALF), jnp.float32),
        pltpu.VMEM_SHARED((ROWS_PAD, HALF), jnp.float32),
        pltpu.SemaphoreType.DMA,
    ],
)
def _segsum_sc(xs_hbm, src_hbm, dst_hbm, zeros_hbm, out_hbm,
               srcv, dstv, rows, acc_sh, sem):
    c = lax.axis_index("c")
    s = lax.axis_index("s")
    pltpu.sync_copy(zeros_hbm, acc_sh.at[pl.ds(s * ROWS_PER_SUB, ROWS_PER_SUB)])
    plsc.subcore_barrier()

    n_chunks = E_PAD // (NS * CHUNK)
    base0 = s * (n_chunks * CHUNK)

    @pl.loop(0, n_chunks)
    def _(j):
        base = base0 + j * CHUNK
        pltpu.sync_copy(src_hbm.at[pl.ds(base, CHUNK)], srcv)
        pltpu.sync_copy(dst_hbm.at[pl.ds(base, CHUNK)], dstv)
        pltpu.async_copy(xs_hbm.at[c].at[srcv], rows, sem).wait()
        pltpu.sync_copy(rows, acc_sh.at[dstv], add=True)

    plsc.subcore_barrier()
    pltpu.sync_copy(
        acc_sh.at[pl.ds(s * ROWS_PER_SUB, ROWS_PER_SUB)],
        out_hbm.at[c].at[pl.ds(s * ROWS_PER_SUB, ROWS_PER_SUB)],
    )


_MBLK = 1000


def _prep_body(deg_ref, x_ref, inv_ref, xs_ref):
    d = deg_ref[0, :, 0:1] + deg_ref[1, :, 0:1]
    iv = lax.rsqrt(jnp.maximum(d, 1.0))
    inv_ref[...] = iv
    xsb = x_ref[...] * iv
    xs_ref[0] = xsb[:, :HALF]
    xs_ref[1] = xsb[:, HALF:]


def _tc_prep(deg2, x):
    return pl.pallas_call(
        _prep_body,
        grid=(N_NODES // _MBLK,),
        in_specs=[
            pl.BlockSpec((NC, _MBLK, 8), lambda i: (0, i, 0)),
            pl.BlockSpec((_MBLK, D), lambda i: (i, 0)),
        ],
        out_specs=[
            pl.BlockSpec((_MBLK, 1), lambda i: (i, 0)),
            pl.BlockSpec((NC, _MBLK, HALF), lambda i: (0, i, 0)),
        ],
        out_shape=[
            jax.ShapeDtypeStruct((N_NODES, 1), jnp.float32),
            jax.ShapeDtypeStruct((NC, N_NODES, HALF), jnp.float32),
        ],
    )(deg2, x)


def _gcn_body(with_relu, agg_ref, hin_ref, inv_ref, w_ref, b_ref, *out_refs):
    a = jnp.concatenate([agg_ref[0], agg_ref[1]], axis=1)
    t = a * inv_ref[...] + hin_ref[...]
    o = jnp.dot(t, w_ref[...], preferred_element_type=jnp.float32) + b_ref[...]
    if with_relu:
        h = jnp.maximum(o, 0.0)
        out_refs[0][...] = h
        hs = h * inv_ref[...]
        out_refs[1][0] = hs[:, :HALF]
        out_refs[1][1] = hs[:, HALF:]
    else:
        out_refs[0][...] = o


def _tc_gcn(agg, hin, inv, w, b, with_relu):
    out_specs = [pl.BlockSpec((_MBLK, D), lambda i: (i, 0))]
    out_shape = [jax.ShapeDtypeStruct((N_NODES, D), jnp.float32)]
    if with_relu:
        out_specs.append(pl.BlockSpec((NC, _MBLK, HALF), lambda i: (0, i, 0)))
        out_shape.append(jax.ShapeDtypeStruct((NC, N_NODES, HALF), jnp.float32))
    res = pl.pallas_call(
        functools.partial(_gcn_body, with_relu),
        grid=(N_NODES // _MBLK,),
        in_specs=[
            pl.BlockSpec((NC, _MBLK, HALF), lambda i: (0, i, 0)),
            pl.BlockSpec((_MBLK, D), lambda i: (i, 0)),
            pl.BlockSpec((_MBLK, 1), lambda i: (i, 0)),
            pl.BlockSpec((D, D), lambda i: (0, 0)),
            pl.BlockSpec((1, D), lambda i: (0, 0)),
        ],
        out_specs=out_specs,
        out_shape=out_shape,
    )(agg, hin, inv, w, b)
    return res if with_relu else res[0]


_KBLK1 = 1280
_NK1 = 10


def _mlp_body(xnn_ref, wm1_ref, bm1_ref, wm2_ref, bm2_ref, z_ref, acc_ref):
    k = pl.program_id(0)

    @pl.when(k == 0)
    def _():
        acc_ref[...] = jnp.zeros_like(acc_ref)

    acc_ref[...] += jnp.dot(xnn_ref[...], wm1_ref[...],
                            preferred_element_type=jnp.float32)

    @pl.when(k == _NK1 - 1)
    def _():
        hm = jnp.maximum(acc_ref[...] + bm1_ref[...], 0.0)
        out = jnp.dot(hm, wm2_ref[...],
                      preferred_element_type=jnp.float32) + bm2_ref[...]
        z_ref[...] = jax.nn.sigmoid(out)


def _tc_mlp(x_nn, Wm1, bm1, Wm2, bm2):
    return pl.pallas_call(
        _mlp_body,
        grid=(_NK1,),
        in_specs=[
            pl.BlockSpec((BATCH, _KBLK1), lambda k: (0, k)),
            pl.BlockSpec((_KBLK1, MLP_HID), lambda k: (k, 0)),
            pl.BlockSpec((1, MLP_HID), lambda k: (0, 0)),
            pl.BlockSpec((MLP_HID, OUT_DIM), lambda k: (0, 0)),
            pl.BlockSpec((1, OUT_DIM), lambda k: (0, 0)),
        ],
        out_specs=pl.BlockSpec((BATCH, OUT_DIM), lambda k: (0, 0)),
        out_shape=jax.ShapeDtypeStruct((BATCH, OUT_DIM), jnp.float32),
        scratch_shapes=[pltpu.VMEM((BATCH, MLP_HID), jnp.float32)],
    )(x_nn, Wm1, bm1, Wm2, bm2)


_KBLK2 = 1792
_NK2 = 7


def _complete_body(xin_ref, wc_ref, z_ref, zc_ref, acc_ref):
    k = pl.program_id(0)

    @pl.when(k == 0)
    def _():
        acc_ref[...] = jnp.zeros_like(acc_ref)

    acc_ref[...] += jnp.dot(xin_ref[...], wc_ref[...],
                            preferred_element_type=jnp.float32)

    @pl.when(k == _NK2 - 1)
    def _():
        zc_ref[...] = acc_ref[...] + z_ref[...]


def _tc_complete(x_input, Wc, z):
    return pl.pallas_call(
        _complete_body,
        grid=(_NK2,),
        in_specs=[
            pl.BlockSpec((BATCH, _KBLK2), lambda k: (0, k)),
            pl.BlockSpec((_KBLK2, OUT_DIM), lambda k: (k, 0)),
            pl.BlockSpec((BATCH, OUT_DIM), lambda k: (0, 0)),
        ],
        out_specs=pl.BlockSpec((BATCH, OUT_DIM), lambda k: (0, 0)),
        out_shape=jax.ShapeDtypeStruct((BATCH, OUT_DIM), jnp.float32),
        scratch_shapes=[pltpu.VMEM((BATCH, OUT_DIM), jnp.float32)],
    )(x_input, Wc, z)


def kernel(x, edge_index, W1, b1, W2, b2, Wm1, bm1, Wm2, bm2, Wc):
    src = edge_index[0]
    dst = edge_index[1]
    npad = E_PAD - N_EDGES
    padi = jnp.arange(npad, dtype=jnp.int32) % 16
    src_p = jnp.concatenate([src, padi])
    dst_p = jnp.concatenate([dst, N_NODES + padi])

    ones8 = jnp.ones((CHUNK, 8), jnp.float32)
    zeros8 = jnp.zeros((ROWS_PER_SUB, 8), jnp.float32)
    zerosH = jnp.zeros((ROWS_PER_SUB, HALF), jnp.float32)

    deg2 = _deg_sc(dst_p, ones8, zeros8)
    inv, xs = _tc_prep(deg2, x)

    agg1 = _segsum_sc(xs, src_p, dst_p, zerosH)
    h, hs = _tc_gcn(agg1, x, inv, W1, b1.reshape(1, D), True)

    agg2 = _segsum_sc(hs, src_p, dst_p, zerosH)
    xg = _tc_gcn(agg2, h, inv, W2, b2.reshape(1, D), False)

    x_nn = xg.reshape(BATCH, N_PER * D)
    x_input = x.reshape(BATCH, N_PER, D)[:, 1:, :].reshape(BATCH, (N_PER - 1) * D)

    z = _tc_mlp(x_nn, Wm1, bm1.reshape(1, MLP_HID), Wm2, bm2.reshape(1, OUT_DIM))
    zc1 = _tc_complete(x_input, Wc, z)
    zc_tensor = jnp.stack([z, zc1], axis=0)
    return (z, zc_tensor, x_input)


# speedup vs baseline: 6.5432x; 6.5432x over previous
import functools

import jax
import jax.numpy as jnp
from jax import lax
from jax.experimental import pallas as pl
from jax.experimental.pallas import tpu as pltpu
from jax.experimental.pallas import tpu_sc as plsc

N_NODES = 10000
N_EDGES = 160000
BATCH = 200
N_PER = 50
D = 256
HALF = 128
MLP_HID = 512
OUT_DIM = 460

NC = 2
NS = 16
E_PAD = 163840
CHUNK = 128
ROWS_PAD = 10240
ROWS_PER_SUB = ROWS_PAD // NS


def _deg_body(dst_hbm, ones_hbm, zeros8_hbm, out_hbm, dstv, onesv, deg_sh, sem):
    c = lax.axis_index("c")
    s = lax.axis_index("s")
    pltpu.sync_copy(zeros8_hbm, deg_sh.at[pl.ds(s * ROWS_PER_SUB, ROWS_PER_SUB)])
    pltpu.sync_copy(ones_hbm, onesv)
    plsc.subcore_barrier()

    n_chunks = E_PAD // (NC * NS * CHUNK)
    base0 = c * (E_PAD // NC) + s * (n_chunks * CHUNK)

    @pl.loop(0, n_chunks)
    def _(j):
        pltpu.sync_copy(dst_hbm.at[pl.ds(base0 + j * CHUNK, CHUNK)], dstv)
        pltpu.sync_copy(onesv, deg_sh.at[dstv], add=True)

    plsc.subcore_barrier()
    pltpu.sync_copy(
        deg_sh.at[pl.ds(s * ROWS_PER_SUB, ROWS_PER_SUB)],
        out_hbm.at[c].at[pl.ds(s * ROWS_PER_SUB, ROWS_PER_SUB)],
    )


def _segsum_body(xs_hbm, src_hbm, dst_hbm, zeros_hbm, out_hbm,
                 srcv, dstv, rows, acc_sh, sem):
    c = lax.axis_index("c")
    s = lax.axis_index("s")
    pltpu.sync_copy(zeros_hbm, acc_sh.at[pl.ds(s * ROWS_PER_SUB, ROWS_PER_SUB)])
    plsc.subcore_barrier()

    n_chunks = E_PAD // (NS * CHUNK)
    base0 = s * (n_chunks * CHUNK)

    @pl.loop(0, n_chunks)
    def _(j):
        base = base0 + j * CHUNK
        pltpu.sync_copy(src_hbm.at[pl.ds(base, CHUNK)], srcv)
        pltpu.sync_copy(dst_hbm.at[pl.ds(base, CHUNK)], dstv)
        pltpu.async_copy(xs_hbm.at[c].at[srcv], rows, sem).wait()
        pltpu.sync_copy(rows, acc_sh.at[dstv], add=True)

    plsc.subcore_barrier()
    pltpu.sync_copy(
        acc_sh.at[pl.ds(s * ROWS_PER_SUB, ROWS_PER_SUB)],
        out_hbm.at[c].at[pl.ds(s * ROWS_PER_SUB, ROWS_PER_SUB)],
    )


@functools.lru_cache(maxsize=None)
def _sc_kernels():
    mesh = plsc.VectorSubcoreMesh(core_axis_name="c", subcore_axis_name="s")
    deg = pl.kernel(
        _deg_body,
        out_type=jax.ShapeDtypeStruct((NC, ROWS_PAD, HALF), jnp.float32),
        mesh=mesh,
        scratch_types=[
            pltpu.VMEM((CHUNK,), jnp.int32),
            pltpu.VMEM((CHUNK, HALF), jnp.float32),
            pltpu.VMEM_SHARED((ROWS_PAD, HALF), jnp.float32),
            pltpu.SemaphoreType.DMA,
        ],
    )
    segsum = pl.kernel(
        _segsum_body,
        out_type=jax.ShapeDtypeStruct((NC, ROWS_PAD, HALF), jnp.float32),
        mesh=mesh,
        scratch_types=[
            pltpu.VMEM((CHUNK,), jnp.int32),
            pltpu.VMEM((CHUNK,), jnp.int32),
            pltpu.VMEM((CHUNK, HALF), jnp.float32),
            pltpu.VMEM_SHARED((ROWS_PAD, HALF), jnp.float32),
            pltpu.SemaphoreType.DMA,
        ],
    )
    return deg, segsum


_MBLK = 1000


def _prep_body(deg_ref, x_ref, inv_ref, xs_ref):
    d = deg_ref[0, :, 0:1] + deg_ref[1, :, 0:1]
    iv = lax.rsqrt(jnp.maximum(d, 1.0))
    inv_ref[...] = iv
    xsb = x_ref[...] * iv
    xs_ref[0] = xsb[:, :HALF]
    xs_ref[1] = xsb[:, HALF:]


def _tc_prep(deg2, x):
    return pl.pallas_call(
        _prep_body,
        grid=(N_NODES // _MBLK,),
        in_specs=[
            pl.BlockSpec((NC, _MBLK, HALF), lambda i: (0, i, 0)),
            pl.BlockSpec((_MBLK, D), lambda i: (i, 0)),
        ],
        out_specs=[
            pl.BlockSpec((_MBLK, 1), lambda i: (i, 0)),
            pl.BlockSpec((NC, _MBLK, HALF), lambda i: (0, i, 0)),
        ],
        out_shape=[
            jax.ShapeDtypeStruct((N_NODES, 1), jnp.float32),
            jax.ShapeDtypeStruct((NC, N_NODES, HALF), jnp.float32),
        ],
    )(deg2, x)


def _gcn_body(with_relu, agg_ref, hin_ref, inv_ref, w_ref, b_ref, *out_refs):
    a = jnp.concatenate([agg_ref[0], agg_ref[1]], axis=1)
    t = a * inv_ref[...] + hin_ref[...]
    o = jnp.dot(t, w_ref[...], preferred_element_type=jnp.float32) + b_ref[...]
    if with_relu:
        h = jnp.maximum(o, 0.0)
        out_refs[0][...] = h
        hs = h * inv_ref[...]
        out_refs[1][0] = hs[:, :HALF]
        out_refs[1][1] = hs[:, HALF:]
    else:
        out_refs[0][...] = o


def _tc_gcn(agg, hin, inv, w, b, with_relu):
    out_specs = [pl.BlockSpec((_MBLK, D), lambda i: (i, 0))]
    out_shape = [jax.ShapeDtypeStruct((N_NODES, D), jnp.float32)]
    if with_relu:
        out_specs.append(pl.BlockSpec((NC, _MBLK, HALF), lambda i: (0, i, 0)))
        out_shape.append(jax.ShapeDtypeStruct((NC, N_NODES, HALF), jnp.float32))
    res = pl.pallas_call(
        functools.partial(_gcn_body, with_relu),
        grid=(N_NODES // _MBLK,),
        in_specs=[
            pl.BlockSpec((NC, _MBLK, HALF), lambda i: (0, i, 0)),
            pl.BlockSpec((_MBLK, D), lambda i: (i, 0)),
            pl.BlockSpec((_MBLK, 1), lambda i: (i, 0)),
            pl.BlockSpec((D, D), lambda i: (0, 0)),
            pl.BlockSpec((1, D), lambda i: (0, 0)),
        ],
        out_specs=out_specs,
        out_shape=out_shape,
    )(agg, hin, inv, w, b)
    return res if with_relu else res[0]


_KBLK1 = 1280
_NK1 = 10


def _mlp_body(xnn_ref, wm1_ref, bm1_ref, wm2_ref, bm2_ref, z_ref, acc_ref):
    k = pl.program_id(0)

    @pl.when(k == 0)
    def _():
        acc_ref[...] = jnp.zeros_like(acc_ref)

    acc_ref[...] += jnp.dot(xnn_ref[...], wm1_ref[...],
                            preferred_element_type=jnp.float32)

    @pl.when(k == _NK1 - 1)
    def _():
        hm = jnp.maximum(acc_ref[...] + bm1_ref[...], 0.0)
        out = jnp.dot(hm, wm2_ref[...],
                      preferred_element_type=jnp.float32) + bm2_ref[...]
        z_ref[...] = jax.nn.sigmoid(out)


def _tc_mlp(x_nn, Wm1, bm1, Wm2, bm2):
    return pl.pallas_call(
        _mlp_body,
        grid=(_NK1,),
        in_specs=[
            pl.BlockSpec((BATCH, _KBLK1), lambda k: (0, k)),
            pl.BlockSpec((_KBLK1, MLP_HID), lambda k: (k, 0)),
            pl.BlockSpec((1, MLP_HID), lambda k: (0, 0)),
            pl.BlockSpec((MLP_HID, OUT_DIM), lambda k: (0, 0)),
            pl.BlockSpec((1, OUT_DIM), lambda k: (0, 0)),
        ],
        out_specs=pl.BlockSpec((BATCH, OUT_DIM), lambda k: (0, 0)),
        out_shape=jax.ShapeDtypeStruct((BATCH, OUT_DIM), jnp.float32),
        scratch_shapes=[pltpu.VMEM((BATCH, MLP_HID), jnp.float32)],
    )(x_nn, Wm1, bm1, Wm2, bm2)


_KBLK2 = 1792
_NK2 = 7


def _complete_body(xin_ref, wc_ref, z_ref, zc_ref, acc_ref):
    k = pl.program_id(0)

    @pl.when(k == 0)
    def _():
        acc_ref[...] = jnp.zeros_like(acc_ref)

    acc_ref[...] += jnp.dot(xin_ref[...], wc_ref[...],
                            preferred_element_type=jnp.float32)

    @pl.when(k == _NK2 - 1)
    def _():
        zc_ref[...] = acc_ref[...] + z_ref[...]


def _tc_complete(x_input, Wc, z):
    return pl.pallas_call(
        _complete_body,
        grid=(_NK2,),
        in_specs=[
            pl.BlockSpec((BATCH, _KBLK2), lambda k: (0, k)),
            pl.BlockSpec((_KBLK2, OUT_DIM), lambda k: (k, 0)),
            pl.BlockSpec((BATCH, OUT_DIM), lambda k: (0, 0)),
        ],
        out_specs=pl.BlockSpec((BATCH, OUT_DIM), lambda k: (0, 0)),
        out_shape=jax.ShapeDtypeStruct((BATCH, OUT_DIM), jnp.float32),
        scratch_shapes=[pltpu.VMEM((BATCH, OUT_DIM), jnp.float32)],
    )(x_input, Wc, z)


def kernel(x, edge_index, W1, b1, W2, b2, Wm1, bm1, Wm2, bm2, Wc):
    src = edge_index[0]
    dst = edge_index[1]
    npad = E_PAD - N_EDGES
    padi = jnp.arange(npad, dtype=jnp.int32) % 16
    src_p = jnp.concatenate([src, padi])
    dst_p = jnp.concatenate([dst, N_NODES + padi])

    ones8 = jnp.ones((CHUNK, HALF), jnp.float32)
    zerosH = jnp.zeros((ROWS_PER_SUB, HALF), jnp.float32)

    _deg_sc, _segsum_sc = _sc_kernels()
    deg2 = _deg_sc(dst_p, ones8, zerosH)
    inv, xs = _tc_prep(deg2, x)

    agg1 = _segsum_sc(xs, src_p, dst_p, zerosH)
    h, hs = _tc_gcn(agg1, x, inv, W1, b1.reshape(1, D), True)

    agg2 = _segsum_sc(hs, src_p, dst_p, zerosH)
    xg = _tc_gcn(agg2, h, inv, W2, b2.reshape(1, D), False)

    x_nn = xg.reshape(BATCH, N_PER * D)
    x_input = x.reshape(BATCH, N_PER, D)[:, 1:, :].reshape(BATCH, (N_PER - 1) * D)

    z = _tc_mlp(x_nn, Wm1, bm1.reshape(1, MLP_HID), Wm2, bm2.reshape(1, OUT_DIM))
    zc1 = _tc_complete(x_input, Wc, z)
    zc_tensor = jnp.stack([z, zc1], axis=0)
    return (z, zc_tensor, x_input)


# speedup vs baseline: 10.1012x; 1.5438x over previous
import functools

import jax
import jax.numpy as jnp
from jax import lax
from jax.experimental import pallas as pl
from jax.experimental.pallas import tpu as pltpu
from jax.experimental.pallas import tpu_sc as plsc

N_NODES = 10000
N_EDGES = 160000
BATCH = 200
N_PER = 50
D = 256
HALF = 128
MLP_HID = 512
OUT_DIM = 460

NC = 2
NS = 16
E_PAD = 163840
CHUNK = 128
ROWS_PAD = 10240
ROWS_PER_SUB = ROWS_PAD // NS


DEG_CH = E_PAD // (NC * NS * CHUNK)
SEG_CH = E_PAD // (NS * CHUNK)


def _deg_body(dst_hbm, ones_hbm, zeros_hbm, out_hbm, dstv2, onesv, deg_sh, ssem):
    c = lax.axis_index("c")
    s = lax.axis_index("s")
    w = c * NS + s
    pltpu.sync_copy(zeros_hbm, deg_sh.at[pl.ds(s * ROWS_PER_SUB, ROWS_PER_SUB)])
    pltpu.sync_copy(ones_hbm, onesv)
    pltpu.sync_copy(dst_hbm.at[w], dstv2)
    plsc.subcore_barrier()

    def start_scatter(j):
        pltpu.async_copy(onesv, deg_sh.at[dstv2.at[j]], ssem, add=True)

    def wait_scatter():
        pltpu.make_async_copy(ones_hbm, onesv, ssem).wait()

    @pl.loop(0, DEG_CH // 4)
    def _(jo):
        for b in range(4):
            j = jo * 4 + b

            @pl.when(j >= 2)
            def _():
                wait_scatter()

            start_scatter(j)

    wait_scatter()
    wait_scatter()
    plsc.subcore_barrier()
    pltpu.sync_copy(
        deg_sh.at[pl.ds(s * ROWS_PER_SUB, ROWS_PER_SUB)],
        out_hbm.at[c].at[pl.ds(s * ROWS_PER_SUB, ROWS_PER_SUB)],
    )


SEG_PH = SEG_CH // 2


def _segsum_body(xs_hbm, src_hbm, dst_hbm, zeros_hbm, out_hbm,
                 srcv2, dstv2, rows2, acc_sh, gsem, ssem):
    c = lax.axis_index("c")
    s = lax.axis_index("s")
    pltpu.sync_copy(zeros_hbm, acc_sh.at[pl.ds(s * ROWS_PER_SUB, ROWS_PER_SUB)])
    plsc.subcore_barrier()

    hbm_dummy = xs_hbm.at[c].at[pl.ds(0, CHUNK)]

    def start_gather(j, b):
        pltpu.async_copy(xs_hbm.at[c].at[srcv2.at[j]], rows2.at[b], gsem)

    def wait_gather():
        pltpu.make_async_copy(hbm_dummy, rows2.at[0], gsem).wait()

    def start_scatter(j, b):
        pltpu.async_copy(rows2.at[b], acc_sh.at[dstv2.at[j]], ssem, add=True)

    def wait_scatter():
        pltpu.make_async_copy(hbm_dummy, rows2.at[0], ssem).wait()

    for ph in range(2):
        pltpu.sync_copy(src_hbm.at[s].at[pl.ds(ph * SEG_PH, SEG_PH)], srcv2)
        pltpu.sync_copy(dst_hbm.at[s].at[pl.ds(ph * SEG_PH, SEG_PH)], dstv2)
        start_gather(0, 0)

        @pl.loop(0, SEG_PH // 4)
        def _(jo):
            for b4 in range(4):
                j = jo * 4 + b4
                b = b4 % 2
                wait_gather()

                @pl.when(j >= 1)
                def _():
                    wait_scatter()

                @pl.when(j + 1 < SEG_PH)
                def _():
                    start_gather(j + 1, (b + 1) % 2)

                start_scatter(j, b)

        wait_scatter()

    plsc.subcore_barrier()
    pltpu.sync_copy(
        acc_sh.at[pl.ds(s * ROWS_PER_SUB, ROWS_PER_SUB)],
        out_hbm.at[c].at[pl.ds(s * ROWS_PER_SUB, ROWS_PER_SUB)],
    )


@functools.lru_cache(maxsize=None)
def _sc_kernels():
    mesh = plsc.VectorSubcoreMesh(core_axis_name="c", subcore_axis_name="s")
    deg = pl.kernel(
        _deg_body,
        out_type=jax.ShapeDtypeStruct((NC, ROWS_PAD, HALF), jnp.float32),
        mesh=mesh,
        scratch_types=[
            pltpu.VMEM((DEG_CH, CHUNK), jnp.int32),
            pltpu.VMEM((CHUNK, HALF), jnp.float32),
            pltpu.VMEM_SHARED((ROWS_PAD, HALF), jnp.float32),
            pltpu.SemaphoreType.DMA,
        ],
    )
    segsum = pl.kernel(
        _segsum_body,
        out_type=jax.ShapeDtypeStruct((NC, ROWS_PAD, HALF), jnp.float32),
        mesh=mesh,
        scratch_types=[
            pltpu.VMEM((SEG_PH, CHUNK), jnp.int32),
            pltpu.VMEM((SEG_PH, CHUNK), jnp.int32),
            pltpu.VMEM((2, CHUNK, HALF), jnp.float32),
            pltpu.VMEM_SHARED((ROWS_PAD, HALF), jnp.float32),
            pltpu.SemaphoreType.DMA,
            pltpu.SemaphoreType.DMA,
        ],
    )
    return deg, segsum


_MBLK = 1000


def _prep_body(deg_ref, x_ref, inv_ref, xs_ref):
    d = deg_ref[0, :, 0:1] + deg_ref[1, :, 0:1]
    iv = lax.rsqrt(jnp.maximum(d, 1.0))
    inv_ref[...] = iv
    xsb = x_ref[...] * iv
    xs_ref[0] = xsb[:, :HALF]
    xs_ref[1] = xsb[:, HALF:]


def _tc_prep(deg2, x):
    return pl.pallas_call(
        _prep_body,
        grid=(N_NODES // _MBLK,),
        in_specs=[
            pl.BlockSpec((NC, _MBLK, HALF), lambda i: (0, i, 0)),
            pl.BlockSpec((_MBLK, D), lambda i: (i, 0)),
        ],
        out_specs=[
            pl.BlockSpec((_MBLK, 1), lambda i: (i, 0)),
            pl.BlockSpec((NC, _MBLK, HALF), lambda i: (0, i, 0)),
        ],
        out_shape=[
            jax.ShapeDtypeStruct((N_NODES, 1), jnp.float32),
            jax.ShapeDtypeStruct((NC, N_NODES, HALF), jnp.float32),
        ],
    )(deg2, x)


def _gcn_body(with_relu, agg_ref, hin_ref, inv_ref, w_ref, b_ref, *out_refs):
    a = jnp.concatenate([agg_ref[0], agg_ref[1]], axis=1)
    t = a * inv_ref[...] + hin_ref[...]
    o = jnp.dot(t, w_ref[...], preferred_element_type=jnp.float32) + b_ref[...]
    if with_relu:
        h = jnp.maximum(o, 0.0)
        out_refs[0][...] = h
        hs = h * inv_ref[...]
        out_refs[1][0] = hs[:, :HALF]
        out_refs[1][1] = hs[:, HALF:]
    else:
        out_refs[0][...] = o


def _tc_gcn(agg, hin, inv, w, b, with_relu):
    out_specs = [pl.BlockSpec((_MBLK, D), lambda i: (i, 0))]
    out_shape = [jax.ShapeDtypeStruct((N_NODES, D), jnp.float32)]
    if with_relu:
        out_specs.append(pl.BlockSpec((NC, _MBLK, HALF), lambda i: (0, i, 0)))
        out_shape.append(jax.ShapeDtypeStruct((NC, N_NODES, HALF), jnp.float32))
    res = pl.pallas_call(
        functools.partial(_gcn_body, with_relu),
        grid=(N_NODES // _MBLK,),
        in_specs=[
            pl.BlockSpec((NC, _MBLK, HALF), lambda i: (0, i, 0)),
            pl.BlockSpec((_MBLK, D), lambda i: (i, 0)),
            pl.BlockSpec((_MBLK, 1), lambda i: (i, 0)),
            pl.BlockSpec((D, D), lambda i: (0, 0)),
            pl.BlockSpec((1, D), lambda i: (0, 0)),
        ],
        out_specs=out_specs,
        out_shape=out_shape,
    )(agg, hin, inv, w, b)
    return res if with_relu else res[0]


_KBLK1 = 1280
_NK1 = 10


def _mlp_body(xnn_ref, wm1_ref, bm1_ref, wm2_ref, bm2_ref, z_ref, acc_ref):
    k = pl.program_id(0)

    @pl.when(k == 0)
    def _():
        acc_ref[...] = jnp.zeros_like(acc_ref)

    acc_ref[...] += jnp.dot(xnn_ref[...], wm1_ref[...],
                            preferred_element_type=jnp.float32)

    @pl.when(k == _NK1 - 1)
    def _():
        hm = jnp.maximum(acc_ref[...] + bm1_ref[...], 0.0)
        out = jnp.dot(hm, wm2_ref[...],
                      preferred_element_type=jnp.float32) + bm2_ref[...]
        z_ref[...] = jax.nn.sigmoid(out)


def _tc_mlp(x_nn, Wm1, bm1, Wm2, bm2):
    return pl.pallas_call(
        _mlp_body,
        grid=(_NK1,),
        in_specs=[
            pl.BlockSpec((BATCH, _KBLK1), lambda k: (0, k)),
            pl.BlockSpec((_KBLK1, MLP_HID), lambda k: (k, 0)),
            pl.BlockSpec((1, MLP_HID), lambda k: (0, 0)),
            pl.BlockSpec((MLP_HID, OUT_DIM), lambda k: (0, 0)),
            pl.BlockSpec((1, OUT_DIM), lambda k: (0, 0)),
        ],
        out_specs=pl.BlockSpec((BATCH, OUT_DIM), lambda k: (0, 0)),
        out_shape=jax.ShapeDtypeStruct((BATCH, OUT_DIM), jnp.float32),
        scratch_shapes=[pltpu.VMEM((BATCH, MLP_HID), jnp.float32)],
    )(x_nn, Wm1, bm1, Wm2, bm2)


_KBLK2 = 1792
_NK2 = 7


def _complete_body(xin_ref, wc_ref, z_ref, zc_ref, acc_ref):
    k = pl.program_id(0)

    @pl.when(k == 0)
    def _():
        acc_ref[...] = jnp.zeros_like(acc_ref)

    acc_ref[...] += jnp.dot(xin_ref[...], wc_ref[...],
                            preferred_element_type=jnp.float32)

    @pl.when(k == _NK2 - 1)
    def _():
        zc_ref[...] = acc_ref[...] + z_ref[...]


def _tc_complete(x_input, Wc, z):
    return pl.pallas_call(
        _complete_body,
        grid=(_NK2,),
        in_specs=[
            pl.BlockSpec((BATCH, _KBLK2), lambda k: (0, k)),
            pl.BlockSpec((_KBLK2, OUT_DIM), lambda k: (k, 0)),
            pl.BlockSpec((BATCH, OUT_DIM), lambda k: (0, 0)),
        ],
        out_specs=pl.BlockSpec((BATCH, OUT_DIM), lambda k: (0, 0)),
        out_shape=jax.ShapeDtypeStruct((BATCH, OUT_DIM), jnp.float32),
        scratch_shapes=[pltpu.VMEM((BATCH, OUT_DIM), jnp.float32)],
    )(x_input, Wc, z)


def kernel(x, edge_index, W1, b1, W2, b2, Wm1, bm1, Wm2, bm2, Wc):
    src = edge_index[0]
    dst = edge_index[1]
    npad = E_PAD - N_EDGES
    padi = jnp.arange(npad, dtype=jnp.int32) % 16
    src_p = jnp.concatenate([src, padi])
    dst_p = jnp.concatenate([dst, N_NODES + padi])
    src3 = src_p.reshape(NS, SEG_CH, CHUNK)
    dst3 = dst_p.reshape(NS, SEG_CH, CHUNK)
    dstd = dst_p.reshape(NC * NS, DEG_CH, CHUNK)

    ones8 = jnp.ones((CHUNK, HALF), jnp.float32)
    zerosH = jnp.zeros((ROWS_PER_SUB, HALF), jnp.float32)

    _deg_sc, _segsum_sc = _sc_kernels()
    deg2 = _deg_sc(dstd, ones8, zerosH)
    inv, xs = _tc_prep(deg2, x)

    agg1 = _segsum_sc(xs, src3, dst3, zerosH)
    h, hs = _tc_gcn(agg1, x, inv, W1, b1.reshape(1, D), True)

    agg2 = _segsum_sc(hs, src3, dst3, zerosH)
    xg = _tc_gcn(agg2, h, inv, W2, b2.reshape(1, D), False)

    x_nn = xg.reshape(BATCH, N_PER * D)
    x_input = x.reshape(BATCH, N_PER, D)[:, 1:, :].reshape(BATCH, (N_PER - 1) * D)

    z = _tc_mlp(x_nn, Wm1, bm1.reshape(1, MLP_HID), Wm2, bm2.reshape(1, OUT_DIM))
    zc1 = _tc_complete(x_input, Wc, z)
    zc_tensor = jnp.stack([z, zc1], axis=0)
    return (z, zc_tensor, x_input)


# speedup vs baseline: 10.2277x; 1.0125x over previous
import functools

import jax
import jax.numpy as jnp
from jax import lax
from jax.experimental import pallas as pl
from jax.experimental.pallas import tpu as pltpu
from jax.experimental.pallas import tpu_sc as plsc

N_NODES = 10000
N_EDGES = 160000
BATCH = 200
N_PER = 50
D = 256
HALF = 128
MLP_HID = 512
OUT_DIM = 460

NC = 2
NS = 16
CHUNK = 128
SEG_CK = 32
E_ROWS = 1280
SEG_ROWS = 5120
ROWS_PAD = 10240
ROWS_PER_SUB = ROWS_PAD // NS

DEG_CH = E_ROWS // (NC * NS)
SEG_CH = SEG_ROWS // NS
SEG_PH = SEG_CH // 8


def _deg_body(dst_hbm, ones_hbm, zeros_hbm, out_hbm, dstv2, onesv, deg_sh, ssem):
    c = lax.axis_index("c")
    s = lax.axis_index("s")
    w = c * NS + s
    pltpu.sync_copy(zeros_hbm, deg_sh.at[pl.ds(s * ROWS_PER_SUB, ROWS_PER_SUB)])
    pltpu.sync_copy(ones_hbm, onesv)
    pltpu.sync_copy(dst_hbm.at[pl.ds(w * DEG_CH, DEG_CH)], dstv2)
    plsc.subcore_barrier()

    def start_scatter(j):
        pltpu.async_copy(onesv, deg_sh.at[dstv2.at[j]], ssem, add=True)

    def wait_scatter():
        pltpu.make_async_copy(ones_hbm, onesv, ssem).wait()

    @pl.loop(0, DEG_CH // 4)
    def _(jo):
        for b in range(4):
            j = jo * 4 + b

            @pl.when(j >= 2)
            def _():
                wait_scatter()

            start_scatter(j)

    wait_scatter()
    wait_scatter()
    plsc.subcore_barrier()
    pltpu.sync_copy(
        deg_sh.at[pl.ds(s * ROWS_PER_SUB, ROWS_PER_SUB)],
        out_hbm.at[c].at[pl.ds(s * ROWS_PER_SUB, ROWS_PER_SUB)],
    )


def _segsum_body(xs_hbm, src_hbm, dst_hbm, zeros_hbm, out_hbm,
                 srcv2, dstv2, rows4, acc_sh, gsem, ssem):
    c = lax.axis_index("c")
    s = lax.axis_index("s")
    pltpu.sync_copy(zeros_hbm, acc_sh.at[pl.ds(s * ROWS_PER_SUB, ROWS_PER_SUB)])
    plsc.subcore_barrier()

    hbm_dummy = xs_hbm.at[c].at[pl.ds(0, SEG_CK)]

    def start_gather(j, b):
        pltpu.async_copy(xs_hbm.at[c].at[srcv2.at[j]], rows4.at[b], gsem)

    def wait_gather():
        pltpu.make_async_copy(hbm_dummy, rows4.at[0], gsem).wait()

    def start_scatter(j, b):
        pltpu.async_copy(rows4.at[b], acc_sh.at[dstv2.at[j]], ssem, add=True)

    def wait_scatter():
        pltpu.make_async_copy(hbm_dummy, rows4.at[0], ssem).wait()

    for ph in range(8):
        base = s * SEG_CH + ph * SEG_PH
        pltpu.sync_copy(src_hbm.at[pl.ds(base, SEG_PH)], srcv2)
        pltpu.sync_copy(dst_hbm.at[pl.ds(base, SEG_PH)], dstv2)
        start_gather(0, 0)
        start_gather(1, 1)
        start_gather(2, 2)

        @pl.loop(0, SEG_PH // 5)
        def _(jo):
            for b in range(5):
                j = jo * 5 + b
                wait_gather()

                @pl.when(j >= 2)
                def _():
                    wait_scatter()

                @pl.when(j + 3 < SEG_PH)
                def _():
                    start_gather(j + 3, (b + 3) % 5)

                start_scatter(j, b)

        wait_scatter()
        wait_scatter()

    plsc.subcore_barrier()
    pltpu.sync_copy(
        acc_sh.at[pl.ds(s * ROWS_PER_SUB, ROWS_PER_SUB)],
        out_hbm.at[c].at[pl.ds(s * ROWS_PER_SUB, ROWS_PER_SUB)],
    )


@functools.lru_cache(maxsize=None)
def _sc_kernels():
    mesh = plsc.VectorSubcoreMesh(core_axis_name="c", subcore_axis_name="s")
    cp = pltpu.CompilerParams(use_tc_tiling_on_sc=True)
    deg = pl.kernel(
        _deg_body,
        compiler_params=cp,
        out_type=jax.ShapeDtypeStruct((NC, ROWS_PAD, HALF), jnp.float32),
        mesh=mesh,
        scratch_types=[
            pltpu.VMEM((DEG_CH, CHUNK), jnp.int32),
            pltpu.VMEM((CHUNK, HALF), jnp.float32),
            pltpu.VMEM_SHARED((ROWS_PAD, HALF), jnp.float32),
            pltpu.SemaphoreType.DMA,
        ],
    )
    segsum = pl.kernel(
        _segsum_body,
        compiler_params=cp,
        out_type=jax.ShapeDtypeStruct((NC, ROWS_PAD, HALF), jnp.float32),
        mesh=mesh,
        scratch_types=[
            pltpu.VMEM((SEG_PH, SEG_CK), jnp.int32),
            pltpu.VMEM((SEG_PH, SEG_CK), jnp.int32),
            pltpu.VMEM((5, SEG_CK, HALF), jnp.float32),
            pltpu.VMEM_SHARED((ROWS_PAD, HALF), jnp.float32),
            pltpu.SemaphoreType.DMA,
            pltpu.SemaphoreType.DMA,
        ],
    )
    return deg, segsum


def _pad_body(ei_ref, pad_ref, src_ref, dst_ref):
    src_ref[0:1248] = ei_ref[0, 0:1248]
    src_ref[1248:E_ROWS] = jnp.concatenate(
        [ei_ref[0, 1248:1250], pad_ref[0]], axis=0)
    dst_ref[0:1248] = ei_ref[1, 0:1248]
    dst_ref[1248:E_ROWS] = jnp.concatenate(
        [ei_ref[1, 1248:1250], pad_ref[1]], axis=0)


def _tc_pad(ei2, pad2):
    return pl.pallas_call(
        _pad_body,
        in_specs=[
            pl.BlockSpec((2, 1250, CHUNK), lambda: (0, 0, 0)),
            pl.BlockSpec((2, 30, CHUNK), lambda: (0, 0, 0)),
        ],
        out_specs=[
            pl.BlockSpec((E_ROWS, CHUNK), lambda: (0, 0)),
            pl.BlockSpec((E_ROWS, CHUNK), lambda: (0, 0)),
        ],
        out_shape=[
            jax.ShapeDtypeStruct((E_ROWS, CHUNK), jnp.int32),
            jax.ShapeDtypeStruct((E_ROWS, CHUNK), jnp.int32),
        ],
    )(ei2, pad2)


_MBLK = 1000


def _prep_body(deg_ref, x_ref, inv_ref, xs_ref):
    d = deg_ref[0, :, 0:1] + deg_ref[1, :, 0:1]
    iv = lax.rsqrt(jnp.maximum(d, 1.0))
    inv_ref[...] = iv
    xsb = x_ref[...] * iv
    xs_ref[0] = xsb[:, :HALF]
    xs_ref[1] = xsb[:, HALF:]


def _tc_prep(deg2, x):
    return pl.pallas_call(
        _prep_body,
        grid=(N_NODES // _MBLK,),
        in_specs=[
            pl.BlockSpec((NC, _MBLK, HALF), lambda i: (0, i, 0)),
            pl.BlockSpec((_MBLK, D), lambda i: (i, 0)),
        ],
        out_specs=[
            pl.BlockSpec((_MBLK, 1), lambda i: (i, 0)),
            pl.BlockSpec((NC, _MBLK, HALF), lambda i: (0, i, 0)),
        ],
        out_shape=[
            jax.ShapeDtypeStruct((N_NODES, 1), jnp.float32),
            jax.ShapeDtypeStruct((NC, N_NODES, HALF), jnp.float32),
        ],
    )(deg2, x)


def _gcn_body(with_relu, agg_ref, hin_ref, inv_ref, w_ref, b_ref, *out_refs):
    a = jnp.concatenate([agg_ref[0], agg_ref[1]], axis=1)
    t = a * inv_ref[...] + hin_ref[...]
    o = jnp.dot(t, w_ref[...], preferred_element_type=jnp.float32) + b_ref[...]
    if with_relu:
        h = jnp.maximum(o, 0.0)
        out_refs[0][...] = h
        hs = h * inv_ref[...]
        out_refs[1][0] = hs[:, :HALF]
        out_refs[1][1] = hs[:, HALF:]
    else:
        out_refs[0][...] = o


def _tc_gcn(agg, hin, inv, w, b, with_relu):
    out_specs = [pl.BlockSpec((_MBLK, D), lambda i: (i, 0))]
    out_shape = [jax.ShapeDtypeStruct((N_NODES, D), jnp.float32)]
    if with_relu:
        out_specs.append(pl.BlockSpec((NC, _MBLK, HALF), lambda i: (0, i, 0)))
        out_shape.append(jax.ShapeDtypeStruct((NC, N_NODES, HALF), jnp.float32))
    res = pl.pallas_call(
        functools.partial(_gcn_body, with_relu),
        grid=(N_NODES // _MBLK,),
        in_specs=[
            pl.BlockSpec((NC, _MBLK, HALF), lambda i: (0, i, 0)),
            pl.BlockSpec((_MBLK, D), lambda i: (i, 0)),
            pl.BlockSpec((_MBLK, 1), lambda i: (i, 0)),
            pl.BlockSpec((D, D), lambda i: (0, 0)),
            pl.BlockSpec((1, D), lambda i: (0, 0)),
        ],
        out_specs=out_specs,
        out_shape=out_shape,
    )(agg, hin, inv, w, b)
    return res if with_relu else res[0]


_KBLK1 = 1280
_NK1 = 10


def _mlp_body(xnn_ref, wm1_ref, bm1_ref, wm2_ref, bm2_ref, cmm_ref,
              z_ref, zc_ref, acc_ref):
    k = pl.program_id(0)

    @pl.when(k == 0)
    def _():
        acc_ref[...] = jnp.zeros_like(acc_ref)

    acc_ref[...] += jnp.dot(xnn_ref[...], wm1_ref[...],
                            preferred_element_type=jnp.float32)

    @pl.when(k == _NK1 - 1)
    def _():
        hm = jnp.maximum(acc_ref[...] + bm1_ref[...], 0.0)
        out = jnp.dot(hm, wm2_ref[...],
                      preferred_element_type=jnp.float32) + bm2_ref[...]
        z = jax.nn.sigmoid(out)
        z_ref[...] = z
        zc_ref[...] = cmm_ref[...] + z


def _tc_mlp(x_nn, Wm1, bm1, Wm2, bm2, cmm):
    return pl.pallas_call(
        _mlp_body,
        grid=(_NK1,),
        in_specs=[
            pl.BlockSpec((BATCH, _KBLK1), lambda k: (0, k)),
            pl.BlockSpec((_KBLK1, MLP_HID), lambda k: (k, 0)),
            pl.BlockSpec((1, MLP_HID), lambda k: (0, 0)),
            pl.BlockSpec((MLP_HID, OUT_DIM), lambda k: (0, 0)),
            pl.BlockSpec((1, OUT_DIM), lambda k: (0, 0)),
            pl.BlockSpec((BATCH, OUT_DIM), lambda k: (0, 0)),
        ],
        out_specs=[
            pl.BlockSpec((BATCH, OUT_DIM), lambda k: (0, 0)),
            pl.BlockSpec((BATCH, OUT_DIM), lambda k: (0, 0)),
        ],
        out_shape=[
            jax.ShapeDtypeStruct((BATCH, OUT_DIM), jnp.float32),
            jax.ShapeDtypeStruct((BATCH, OUT_DIM), jnp.float32),
        ],
        scratch_shapes=[pltpu.VMEM((BATCH, MLP_HID), jnp.float32)],
    )(x_nn, Wm1, bm1, Wm2, bm2, cmm)


_KBLK2 = 1792
_NK2 = 7


def _complete_body(xin_ref, wc_ref, zc_ref, acc_ref):
    k = pl.program_id(0)

    @pl.when(k == 0)
    def _():
        acc_ref[...] = jnp.zeros_like(acc_ref)

    acc_ref[...] += jnp.dot(xin_ref[...], wc_ref[...],
                            preferred_element_type=jnp.float32)

    @pl.when(k == _NK2 - 1)
    def _():
        zc_ref[...] = acc_ref[...]


def _tc_complete(x_input, Wc):
    return pl.pallas_call(
        _complete_body,
        grid=(_NK2,),
        in_specs=[
            pl.BlockSpec((BATCH, _KBLK2), lambda k: (0, k)),
            pl.BlockSpec((_KBLK2, OUT_DIM), lambda k: (k, 0)),
        ],
        out_specs=pl.BlockSpec((BATCH, OUT_DIM), lambda k: (0, 0)),
        out_shape=jax.ShapeDtypeStruct((BATCH, OUT_DIM), jnp.float32),
        scratch_shapes=[pltpu.VMEM((BATCH, OUT_DIM), jnp.float32)],
    )(x_input, Wc)


def kernel(x, edge_index, W1, b1, W2, b2, Wm1, bm1, Wm2, bm2, Wc):
    ei2 = edge_index.reshape(2, 1250, CHUNK)
    padi = jnp.arange(30 * CHUNK, dtype=jnp.int32) % 16
    pad2 = jnp.stack([padi, N_NODES + padi]).reshape(2, 30, CHUNK)
    src2d, dst2d = _tc_pad(ei2, pad2)

    ones8 = jnp.ones((CHUNK, HALF), jnp.float32)
    zerosH = jnp.zeros((ROWS_PER_SUB, HALF), jnp.float32)

    x_input = x.reshape(BATCH, N_PER, D)[:, 1:, :].reshape(BATCH, (N_PER - 1) * D)
    cmm = _tc_complete(x_input, Wc)

    _deg_sc, _segsum_sc = _sc_kernels()
    deg2 = _deg_sc(dst2d, ones8, zerosH)
    inv, xs = _tc_prep(deg2, x)

    src2d_64 = src2d.reshape(SEG_ROWS, SEG_CK)
    dst2d_64 = dst2d.reshape(SEG_ROWS, SEG_CK)
    agg1 = _segsum_sc(xs, src2d_64, dst2d_64, zerosH)
    h, hs = _tc_gcn(agg1, x, inv, W1, b1.reshape(1, D), True)

    agg2 = _segsum_sc(hs, src2d_64, dst2d_64, zerosH)
    xg = _tc_gcn(agg2, h, inv, W2, b2.reshape(1, D), False)

    x_nn = xg.reshape(BATCH, N_PER * D)
    z, zc1 = _tc_mlp(x_nn, Wm1, bm1.reshape(1, MLP_HID), Wm2,
                     bm2.reshape(1, OUT_DIM), cmm)
    zc_tensor = jnp.stack([z, zc1], axis=0)
    return (z, zc_tensor, x_input)


# speedup vs baseline: 10.9861x; 1.0742x over previous
import functools

import jax
import jax.numpy as jnp
from jax import lax
from jax.experimental import pallas as pl
from jax.experimental.pallas import tpu as pltpu
from jax.experimental.pallas import tpu_sc as plsc

N_NODES = 10000
N_EDGES = 160000
BATCH = 200
N_PER = 50
D = 256
HALF = 128
MLP_HID = 512
OUT_DIM = 460

NC = 2
NS = 16
CHUNK = 128
SEG_CK = 64
E_ROWS = 1280
SEG_ROWS = 2560
ROWS_PAD = 10240
ROWS_PER_SUB = ROWS_PAD // NS

DEG_CH = E_ROWS // (NC * NS)
SEG_CH = SEG_ROWS // NS
SEG_PH = SEG_CH // 4


def _deg_body(dst_hbm, ones_hbm, zeros_hbm, out_hbm, dstv2, onesv, deg_sh, ssem):
    c = lax.axis_index("c")
    s = lax.axis_index("s")
    w = c * NS + s
    pltpu.sync_copy(zeros_hbm, deg_sh.at[pl.ds(s * ROWS_PER_SUB, ROWS_PER_SUB)])
    pltpu.sync_copy(ones_hbm, onesv)
    pltpu.sync_copy(dst_hbm.at[pl.ds(w * DEG_CH, DEG_CH)], dstv2)
    plsc.subcore_barrier()

    def start_scatter(j):
        pltpu.async_copy(onesv, deg_sh.at[dstv2.at[j]], ssem, add=True)

    def wait_scatter():
        pltpu.make_async_copy(ones_hbm, onesv, ssem).wait()

    @pl.loop(0, DEG_CH // 4)
    def _(jo):
        for b in range(4):
            j = jo * 4 + b

            @pl.when(j >= 4)
            def _():
                wait_scatter()

            start_scatter(j)

    wait_scatter()
    wait_scatter()
    wait_scatter()
    wait_scatter()
    plsc.subcore_barrier()
    pltpu.sync_copy(
        deg_sh.at[pl.ds(s * ROWS_PER_SUB, ROWS_PER_SUB)],
        out_hbm.at[c].at[pl.ds(s * ROWS_PER_SUB, ROWS_PER_SUB)],
    )


def _segsum_body(xs_hbm, src_hbm, dst_hbm, zeros_hbm, out_hbm,
                 srcv2, dstv2, rows4, acc_sh, gsem, ssem):
    c = lax.axis_index("c")
    s = lax.axis_index("s")
    pltpu.sync_copy(zeros_hbm, acc_sh.at[pl.ds(s * ROWS_PER_SUB, ROWS_PER_SUB)])
    plsc.subcore_barrier()

    hbm_dummy = xs_hbm.at[c].at[pl.ds(0, SEG_CK)]

    def start_gather(j, b):
        pltpu.async_copy(xs_hbm.at[c].at[srcv2.at[j]], rows4.at[b], gsem)

    def wait_gather():
        pltpu.make_async_copy(hbm_dummy, rows4.at[0], gsem).wait()

    def start_scatter(j, b):
        pltpu.async_copy(rows4.at[b], acc_sh.at[dstv2.at[j]], ssem, add=True)

    def wait_scatter():
        pltpu.make_async_copy(hbm_dummy, rows4.at[0], ssem).wait()

    for ph in range(4):
        base = s * SEG_CH + ph * SEG_PH
        pltpu.sync_copy(src_hbm.at[pl.ds(base, SEG_PH)], srcv2)
        pltpu.sync_copy(dst_hbm.at[pl.ds(base, SEG_PH)], dstv2)
        start_gather(0, 0)
        start_gather(1, 1)

        @pl.loop(0, SEG_PH // 4)
        def _(jo):
            for b in range(4):
                j = jo * 4 + b
                wait_gather()

                @pl.when(j >= 2)
                def _():
                    wait_scatter()

                @pl.when(j + 2 < SEG_PH)
                def _():
                    start_gather(j + 2, (b + 2) % 4)

                start_scatter(j, b)

        wait_scatter()
        wait_scatter()

    plsc.subcore_barrier()
    pltpu.sync_copy(
        acc_sh.at[pl.ds(s * ROWS_PER_SUB, ROWS_PER_SUB)],
        out_hbm.at[c].at[pl.ds(s * ROWS_PER_SUB, ROWS_PER_SUB)],
    )


@functools.lru_cache(maxsize=None)
def _sc_kernels():
    mesh = plsc.VectorSubcoreMesh(core_axis_name="c", subcore_axis_name="s")
    cp = pltpu.CompilerParams(use_tc_tiling_on_sc=True)
    deg = pl.kernel(
        _deg_body,
        compiler_params=cp,
        out_type=jax.ShapeDtypeStruct((NC, ROWS_PAD, HALF), jnp.float32),
        mesh=mesh,
        scratch_types=[
            pltpu.VMEM((DEG_CH, CHUNK), jnp.int32),
            pltpu.VMEM((CHUNK, HALF), jnp.float32),
            pltpu.VMEM_SHARED((ROWS_PAD, HALF), jnp.float32),
            pltpu.SemaphoreType.DMA,
        ],
    )
    segsum = pl.kernel(
        _segsum_body,
        compiler_params=cp,
        out_type=jax.ShapeDtypeStruct((NC, ROWS_PAD, HALF), jnp.float32),
        mesh=mesh,
        scratch_types=[
            pltpu.VMEM((SEG_PH, SEG_CK), jnp.int32),
            pltpu.VMEM((SEG_PH, SEG_CK), jnp.int32),
            pltpu.VMEM((4, SEG_CK, HALF), jnp.float32),
            pltpu.VMEM_SHARED((ROWS_PAD, HALF), jnp.float32),
            pltpu.SemaphoreType.DMA,
            pltpu.SemaphoreType.DMA,
        ],
    )
    return deg, segsum


def _pad_body(ei_ref, pad_ref, src_ref, dst_ref):
    src_ref[0:1248] = ei_ref[0, 0:1248]
    src_ref[1248:E_ROWS] = jnp.concatenate(
        [ei_ref[0, 1248:1250], pad_ref[0]], axis=0)
    dst_ref[0:1248] = ei_ref[1, 0:1248]
    dst_ref[1248:E_ROWS] = jnp.concatenate(
        [ei_ref[1, 1248:1250], pad_ref[1]], axis=0)


def _tc_pad(ei2, pad2):
    return pl.pallas_call(
        _pad_body,
        in_specs=[
            pl.BlockSpec((2, 1250, CHUNK), lambda: (0, 0, 0)),
            pl.BlockSpec((2, 30, CHUNK), lambda: (0, 0, 0)),
        ],
        out_specs=[
            pl.BlockSpec((E_ROWS, CHUNK), lambda: (0, 0)),
            pl.BlockSpec((E_ROWS, CHUNK), lambda: (0, 0)),
        ],
        out_shape=[
            jax.ShapeDtypeStruct((E_ROWS, CHUNK), jnp.int32),
            jax.ShapeDtypeStruct((E_ROWS, CHUNK), jnp.int32),
        ],
    )(ei2, pad2)


_MBLK = 1000


def _prep_body(deg_ref, x_ref, inv_ref, xs_ref):
    d = deg_ref[0, :, 0:1] + deg_ref[1, :, 0:1]
    iv = lax.rsqrt(jnp.maximum(d, 1.0))
    inv_ref[...] = iv
    xsb = x_ref[...] * iv
    xs_ref[0] = xsb[:, :HALF]
    xs_ref[1] = xsb[:, HALF:]


def _tc_prep(deg2, x):
    return pl.pallas_call(
        _prep_body,
        grid=(N_NODES // _MBLK,),
        in_specs=[
            pl.BlockSpec((NC, _MBLK, HALF), lambda i: (0, i, 0)),
            pl.BlockSpec((_MBLK, D), lambda i: (i, 0)),
        ],
        out_specs=[
            pl.BlockSpec((_MBLK, 1), lambda i: (i, 0)),
            pl.BlockSpec((NC, _MBLK, HALF), lambda i: (0, i, 0)),
        ],
        out_shape=[
            jax.ShapeDtypeStruct((N_NODES, 1), jnp.float32),
            jax.ShapeDtypeStruct((NC, N_NODES, HALF), jnp.float32),
        ],
    )(deg2, x)


def _gcn_body(with_relu, agg_ref, hin_ref, inv_ref, w_ref, b_ref, *out_refs):
    a = jnp.concatenate([agg_ref[0], agg_ref[1]], axis=1)
    t = a * inv_ref[...] + hin_ref[...]
    o = jnp.dot(t, w_ref[...], preferred_element_type=jnp.float32) + b_ref[...]
    if with_relu:
        h = jnp.maximum(o, 0.0)
        out_refs[0][...] = h
        hs = h * inv_ref[...]
        out_refs[1][0] = hs[:, :HALF]
        out_refs[1][1] = hs[:, HALF:]
    else:
        out_refs[0][...] = o


def _tc_gcn(agg, hin, inv, w, b, with_relu):
    out_specs = [pl.BlockSpec((_MBLK, D), lambda i: (i, 0))]
    out_shape = [jax.ShapeDtypeStruct((N_NODES, D), jnp.float32)]
    if with_relu:
        out_specs.append(pl.BlockSpec((NC, _MBLK, HALF), lambda i: (0, i, 0)))
        out_shape.append(jax.ShapeDtypeStruct((NC, N_NODES, HALF), jnp.float32))
    res = pl.pallas_call(
        functools.partial(_gcn_body, with_relu),
        grid=(N_NODES // _MBLK,),
        in_specs=[
            pl.BlockSpec((NC, _MBLK, HALF), lambda i: (0, i, 0)),
            pl.BlockSpec((_MBLK, D), lambda i: (i, 0)),
            pl.BlockSpec((_MBLK, 1), lambda i: (i, 0)),
            pl.BlockSpec((D, D), lambda i: (0, 0)),
            pl.BlockSpec((1, D), lambda i: (0, 0)),
        ],
        out_specs=out_specs,
        out_shape=out_shape,
    )(agg, hin, inv, w, b)
    return res if with_relu else res[0]


_KBLK1 = 1280
_NK1 = 10


def _mlp_body(xnn_ref, wm1_ref, bm1_ref, wm2_ref, bm2_ref, cmm_ref,
              z_ref, zc_ref, acc_ref):
    k = pl.program_id(0)

    @pl.when(k == 0)
    def _():
        acc_ref[...] = jnp.zeros_like(acc_ref)

    acc_ref[...] += jnp.dot(xnn_ref[...], wm1_ref[...],
                            preferred_element_type=jnp.float32)

    @pl.when(k == _NK1 - 1)
    def _():
        hm = jnp.maximum(acc_ref[...] + bm1_ref[...], 0.0)
        out = jnp.dot(hm, wm2_ref[...],
                      preferred_element_type=jnp.float32) + bm2_ref[...]
        z = jax.nn.sigmoid(out)
        z_ref[...] = z
        zc_ref[...] = cmm_ref[...] + z


def _tc_mlp(x_nn, Wm1, bm1, Wm2, bm2, cmm):
    return pl.pallas_call(
        _mlp_body,
        grid=(_NK1,),
        in_specs=[
            pl.BlockSpec((BATCH, _KBLK1), lambda k: (0, k)),
            pl.BlockSpec((_KBLK1, MLP_HID), lambda k: (k, 0)),
            pl.BlockSpec((1, MLP_HID), lambda k: (0, 0)),
            pl.BlockSpec((MLP_HID, OUT_DIM), lambda k: (0, 0)),
            pl.BlockSpec((1, OUT_DIM), lambda k: (0, 0)),
            pl.BlockSpec((BATCH, OUT_DIM), lambda k: (0, 0)),
        ],
        out_specs=[
            pl.BlockSpec((BATCH, OUT_DIM), lambda k: (0, 0)),
            pl.BlockSpec((BATCH, OUT_DIM), lambda k: (0, 0)),
        ],
        out_shape=[
            jax.ShapeDtypeStruct((BATCH, OUT_DIM), jnp.float32),
            jax.ShapeDtypeStruct((BATCH, OUT_DIM), jnp.float32),
        ],
        scratch_shapes=[pltpu.VMEM((BATCH, MLP_HID), jnp.float32)],
    )(x_nn, Wm1, bm1, Wm2, bm2, cmm)


_KBLK2 = 1792
_NK2 = 7


def _complete_body(xin_ref, wc_ref, zc_ref, acc_ref):
    k = pl.program_id(0)

    @pl.when(k == 0)
    def _():
        acc_ref[...] = jnp.zeros_like(acc_ref)

    acc_ref[...] += jnp.dot(xin_ref[...], wc_ref[...],
                            preferred_element_type=jnp.float32)

    @pl.when(k == _NK2 - 1)
    def _():
        zc_ref[...] = acc_ref[...]


def _tc_complete(x_input, Wc):
    return pl.pallas_call(
        _complete_body,
        grid=(_NK2,),
        in_specs=[
            pl.BlockSpec((BATCH, _KBLK2), lambda k: (0, k)),
            pl.BlockSpec((_KBLK2, OUT_DIM), lambda k: (k, 0)),
        ],
        out_specs=pl.BlockSpec((BATCH, OUT_DIM), lambda k: (0, 0)),
        out_shape=jax.ShapeDtypeStruct((BATCH, OUT_DIM), jnp.float32),
        scratch_shapes=[pltpu.VMEM((BATCH, OUT_DIM), jnp.float32)],
    )(x_input, Wc)


def kernel(x, edge_index, W1, b1, W2, b2, Wm1, bm1, Wm2, bm2, Wc):
    ei2 = edge_index.reshape(2, 1250, CHUNK)
    padi = jnp.arange(30 * CHUNK, dtype=jnp.int32) % 16
    pad2 = jnp.stack([padi, N_NODES + padi]).reshape(2, 30, CHUNK)
    src2d, dst2d = _tc_pad(ei2, pad2)

    ones8 = jnp.ones((CHUNK, HALF), jnp.float32)
    zerosH = jnp.zeros((ROWS_PER_SUB, HALF), jnp.float32)

    x_input = x.reshape(BATCH, N_PER, D)[:, 1:, :].reshape(BATCH, (N_PER - 1) * D)
    cmm = _tc_complete(x_input, Wc)

    _deg_sc, _segsum_sc = _sc_kernels()
    deg2 = _deg_sc(dst2d, ones8, zerosH)
    inv, xs = _tc_prep(deg2, x)

    src2d_64 = src2d.reshape(SEG_ROWS, SEG_CK)
    dst2d_64 = dst2d.reshape(SEG_ROWS, SEG_CK)
    agg1 = _segsum_sc(xs, src2d_64, dst2d_64, zerosH)
    h, hs = _tc_gcn(agg1, x, inv, W1, b1.reshape(1, D), True)

    agg2 = _segsum_sc(hs, src2d_64, dst2d_64, zerosH)
    xg = _tc_gcn(agg2, h, inv, W2, b2.reshape(1, D), False)

    x_nn = xg.reshape(BATCH, N_PER * D)
    z, zc1 = _tc_mlp(x_nn, Wm1, bm1.reshape(1, MLP_HID), Wm2,
                     bm2.reshape(1, OUT_DIM), cmm)
    zc_tensor = jnp.stack([z, zc1], axis=0)
    return (z, zc_tensor, x_input)
